# baseline (speedup 1.0000x reference)
import jax
import jax.numpy as jnp
from jax.experimental import pallas as pl

_GROUP_ORDER = 16
_FACTOR = 2
_SUB = _GROUP_ORDER // _FACTOR
_F = 96
_SPLIT = 2


def _copy_body(in_ref, out_ref):
    out_ref[...] = in_ref[...]


def kernel(x):
    B, C, H, W = x.shape
    bs = B // _SPLIT
    outs = []
    for p in range(_SPLIT):
        xv = x[p * bs:(p + 1) * bs].reshape(bs, _GROUP_ORDER, _F, H, W)
        o = pl.pallas_call(
            _copy_body,
            grid=(bs, _SUB),
            in_specs=[
                pl.BlockSpec((1, 1, _F, H, W),
                             lambda b, g: (b, _FACTOR * g, 0, 0, 0))
            ],
            out_specs=pl.BlockSpec((1, 1, _F, H, W),
                                   lambda b, g: (b, g, 0, 0, 0)),
            out_shape=jax.ShapeDtypeStruct((bs, _SUB, _F, H, W), jnp.float32),
        )(xv)
        outs.append(o)
    out = jnp.concatenate(outs, axis=0)
    return out.reshape(B, _SUB * _F, H, W)
